# baseline (speedup 1.0000x reference)
import functools

import jax
import jax.numpy as jnp
from jax import lax
from jax.experimental import pallas as pl
from jax.experimental.pallas import tpu as pltpu
from jax.experimental.pallas import tpu_sc as plsc

N = 10000
E = 320000
D = 128
R = 65
NC = 2
NS = 16
NT = NC * NS
KB = 80
EPT = E // NS
NBLK2 = EPT // KB
NBLK = NBLK2 // NC
CNTP = 651264
STRIPE = CNTP // NS
ZCH = 6784
NH = 5120
NPH = 5248
ZROWS = NPH // NS
WROWS = NH // NS

_mesh = plsc.VectorSubcoreMesh(core_axis_name="c", subcore_axis_name="s")


@functools.partial(
    pl.kernel,
    out_type=jax.ShapeDtypeStruct((NC, 1, CNTP), jnp.float32),
    mesh=_mesh,
    scratch_types=[
        pltpu.VMEM((NBLK, KB), jnp.int32),
        pltpu.VMEM((NBLK, KB), jnp.int32),
        pltpu.VMEM((NBLK, KB), jnp.int32),
        pltpu.VMEM((KB,), jnp.float32),
        pltpu.VMEM((ZCH,), jnp.float32),
        pltpu.VMEM_SHARED((CNTP,), jnp.float32),
    ],
)
def _sc_counts(d3, t3, out, dv, tv, civ, ones, zc, cnt_sh):
    c = lax.axis_index("c")
    s = lax.axis_index("s")
    wid = c * NS + s

    def fill_ones(i, carry):
        ones[pl.ds(i * 16, 16)] = jnp.full((16,), 1.0, jnp.float32)
        return carry

    lax.fori_loop(0, KB // 16, fill_ones, 0)

    def fill_z(i, carry):
        zc[pl.ds(i * 16, 16)] = jnp.zeros((16,), jnp.float32)
        return carry

    lax.fori_loop(0, ZCH // 16, fill_z, 0)

    base0 = s * STRIPE
    for k in range(STRIPE // ZCH):
        pltpu.sync_copy(zc, cnt_sh.at[pl.ds(base0 + k * ZCH, ZCH)])
    plsc.subcore_barrier()

    pltpu.sync_copy(d3.at[wid], dv)
    pltpu.sync_copy(t3.at[wid], tv)

    def mkci(i, carry):
        j = i // 5
        k = i % 5
        sl = pl.ds(k * 16, 16)
        civ[j, sl] = tv[j, sl] * N + dv[j, sl]
        return carry

    lax.fori_loop(0, NBLK * 5, mkci, 0)

    def scat(j, carry):
        pltpu.sync_copy(ones, cnt_sh.at[civ.at[j]], add=True)
        return carry

    lax.fori_loop(0, NBLK, scat, 0)
    plsc.subcore_barrier()

    pltpu.sync_copy(cnt_sh.at[pl.ds(base0, STRIPE)],
                    out.at[c, 0, pl.ds(base0, STRIPE)])


@functools.partial(
    pl.kernel,
    out_type=jax.ShapeDtypeStruct((NC, NH, D), jnp.float32),
    mesh=_mesh,
    scratch_types=[
        pltpu.VMEM((EPT,), jnp.int32),
        pltpu.VMEM((EPT,), jnp.int32),
        pltpu.VMEM((KB,), jnp.int32),
        pltpu.VMEM((KB,), jnp.int32),
        pltpu.VMEM((KB,), jnp.int32),
        pltpu.VMEM((KB,), jnp.int32),
        pltpu.VMEM((KB + 16,), jnp.float32),
        pltpu.VMEM((KB + 16,), jnp.float32),
        pltpu.VMEM((KB + 16,), jnp.float32),
        pltpu.VMEM((KB + 16,), jnp.float32),
        pltpu.VMEM((KB, D), jnp.float32),
        pltpu.VMEM((KB, D), jnp.float32),
        pltpu.VMEM((KB, D), jnp.float32),
        pltpu.VMEM((KB, D), jnp.float32),
        pltpu.VMEM_SHARED((NPH, D), jnp.float32),
        pltpu.SemaphoreType.DMA,
        pltpu.SemaphoreType.DMA,
        pltpu.SemaphoreType.DMA,
        pltpu.SemaphoreType.DMA,
        pltpu.SemaphoreType.DMA,
        pltpu.SemaphoreType.DMA,
        pltpu.SemaphoreType.DMA,
        pltpu.SemaphoreType.DMA,
        pltpu.SemaphoreType.DMA,
        pltpu.SemaphoreType.DMA,
        pltpu.SemaphoreType.DMA,
        pltpu.SemaphoreType.DMA,
    ],
)
def _sc_scatter(s1, dm, t1, htab, inv, out, gv, civ,
                dvb0, dvb1, dvb2, dvb3, wvb0, wvb1, wvb2, wvb3,
                rows0, rows1, rows2, rows3, acc_sh,
                semg0, semg1, semg2, semg3, semw0, semw1, semw2, semw3,
                sems0, sems1, sems2, sems3):
    c = lax.axis_index("c")
    s = lax.axis_index("s")
    r0z = s * ZROWS
    r0w = s * WROWS

    pltpu.sync_copy(t1.at[s], civ)
    pltpu.sync_copy(dm.at[s], gv)
    lo = c * NH

    def mkciv(i, carry):
        sl = pl.ds(i * 16, 16)
        civ[sl] = civ[sl] * N + gv[sl]
        return carry

    lax.fori_loop(0, EPT // 16, mkciv, 0)
    pltpu.sync_copy(s1.at[s], gv)

    def mkgv(i, carry):
        sl = pl.ds(i * 16, 16)
        cc = civ[sl]
        gv[sl] = cc - cc % N + gv[sl]
        return carry

    lax.fori_loop(0, EPT // 16, mkgv, 0)

    def zrow(i, carry):
        rows0[i // 8, pl.ds((i % 8) * 16, 16)] = jnp.zeros((16,), jnp.float32)
        return carry

    lax.fori_loop(0, KB * 8, zrow, 0)
    for k in range(ZROWS // KB):
        pltpu.sync_copy(rows0, acc_sh.at[pl.ds(r0z + k * KB, KB)])
    pltpu.sync_copy(rows0.at[pl.ds(0, ZROWS % KB)],
                    acc_sh.at[pl.ds(r0z + (ZROWS // KB) * KB, ZROWS % KB)])
    plsc.subcore_barrier()

    bufs = ((rows0, wvb0, dvb0, semg0, semw0, sems0),
            (rows1, wvb1, dvb1, semg1, semw1, sems1),
            (rows2, wvb2, dvb2, semg2, semw2, sems2),
            (rows3, wvb3, dvb3, semg3, semw3, sems3))

    def _issue_gather(j, p):
        rw, wv, _, sg, sw, _ = bufs[p]
        pltpu.async_copy(htab.at[gv.at[pl.ds(j * KB, KB)]], rw, sg)
        pltpu.async_copy(inv.at[civ.at[pl.ds(j * KB, KB)]],
                         wv.at[pl.ds(0, KB)], sw)

    def _wait_gather(j, p):
        rw, wv, _, sg, sw, _ = bufs[p]
        pltpu.make_async_copy(htab.at[gv.at[pl.ds(j * KB, KB)]], rw, sg).wait()
        pltpu.make_async_copy(inv.at[civ.at[pl.ds(j * KB, KB)]],
                              wv.at[pl.ds(0, KB)], sw).wait()

    def _issue_scatter(j, p):
        rw, _, db, _, _, ss = bufs[p]
        pltpu.async_copy(rw, acc_sh.at[db], ss, add=True)

    def _wait_scatter(j, p):
        rw, _, db, _, _, ss = bufs[p]
        pltpu.make_async_copy(rw, acc_sh.at[db], ss).wait()

    def _mkdvb(j, p):
        db = bufs[p][2]
        for k in range(KB // 16):
            sl = pl.ds(j * KB + k * 16, 16)
            local = civ[sl] % N - lo
            ok = (local >= 0) & (local < NH)
            db[pl.ds(k * 16, 16)] = jnp.where(ok, local, NH)

    def _scale(j, p):
        rw, wv = bufs[p][0], bufs[p][1]

        def body(g, inner):
            wch = wv[pl.ds(g * 16, 16)]
            for l in range(16):
                w = wch[l]
                e = g * 16 + l
                for k in range(8):
                    sl = pl.ds(k * 16, 16)
                    rw[e, sl] = rw[e, sl] * w
            return inner

        lax.fori_loop(0, KB // 16, body, 0)

    def _step(j, guard_ws, guard_ig):
        p = j % 4
        _wait_gather(j, p)
        if guard_ws:
            _wait_scatter(j - 2, (j + 2) % 4)
        if guard_ig:
            _issue_gather(j + 2, (j + 2) % 4)
        _mkdvb(j, p)
        _scale(j, p)
        _issue_scatter(j, p)

    _issue_gather(0, 0)
    _issue_gather(1, 1)
    _step(0, False, True)
    _step(1, False, True)

    def quad(j4, carry):
        base = 2 + 4 * j4
        for q in range(4):
            j = base + q
            p = (2 + q) % 4
            _wait_gather(j, p)
            _wait_scatter(j - 2, q)
            _issue_gather(j + 2, q)
            _mkdvb(j, p)
            _scale(j, p)
            _issue_scatter(j, p)
        return carry

    lax.fori_loop(0, (NBLK2 - 6) // 4, quad, 0)
    _step(246, True, True)
    _step(247, True, True)
    _step(248, True, False)
    _step(249, True, False)
    _wait_scatter(248, 0)
    _wait_scatter(249, 1)
    plsc.subcore_barrier()
    pltpu.sync_copy(acc_sh.at[pl.ds(r0w, WROWS)],
                    out.at[c, pl.ds(r0w, WROWS)])


def _inv_body(c_ref, o_ref):
    o_ref[...] = 1.0 / jnp.maximum(c_ref[0] + c_ref[1], 1.0)


_tc_inv = pl.pallas_call(
    _inv_body,
    out_shape=jax.ShapeDtypeStruct((CNTP // 128, 128), jnp.float32),
)


def _root_body(h_ref, wr_ref, ws_ref, b_ref, o_ref):
    w = wr_ref[...] + ws_ref[...]
    o_ref[...] = jnp.dot(h_ref[...], w, preferred_element_type=jnp.float32) + b_ref[...]


_tc_root = pl.pallas_call(
    _root_body,
    out_shape=jax.ShapeDtypeStruct((N, D), jnp.float32),
)


def _h_body(h_ref, w_ref, o_ref):
    o_ref[...] = jnp.dot(h_ref[...], w_ref[0], preferred_element_type=jnp.float32)


_tc_h = pl.pallas_call(
    _h_body,
    grid=(5, R),
    in_specs=[
        pl.BlockSpec((N // 5, D), lambda i, r: (i, 0)),
        pl.BlockSpec((1, D, D), lambda i, r: (r, 0, 0)),
    ],
    out_specs=pl.BlockSpec((N // 5, D), lambda i, r: (r * 5 + i, 0)),
    out_shape=jax.ShapeDtypeStruct((R * N, D), jnp.float32),
)


def _comb_body(r_ref, a_ref, o_ref):
    agg = jnp.concatenate([a_ref[0], a_ref[1]], axis=0)[:N]
    o_ref[...] = jnp.maximum(r_ref[...] + agg, 0.0)


_tc_combine = pl.pallas_call(
    _comb_body,
    out_shape=jax.ShapeDtypeStruct((N, D), jnp.float32),
)


def kernel(x, edge_index, edge_type, W_rel1, W_root1, b1, W_self1,
           W_rel2, W_root2, b2, W_self2):
    s1 = edge_index[0].reshape(NS, EPT)
    dmr = edge_index[1].reshape(NS, EPT)
    t1 = edge_type.reshape(NS, EPT)

    cnt2 = _sc_counts(edge_index[1].reshape(NT, NBLK, KB),
                      edge_type.reshape(NT, NBLK, KB))
    inv = _tc_inv(cnt2.reshape(NC, CNTP // 128, 128)).reshape(CNTP)

    root1 = _tc_root(x, W_root1, W_self1, b1.reshape(1, D))
    htab1 = _tc_h(x, W_rel1)
    acc1 = _sc_scatter(s1, dmr, t1, htab1, inv)
    h1 = _tc_combine(root1, acc1)

    root2 = _tc_root(h1, W_root2, W_self2, b2.reshape(1, D))
    htab2 = _tc_h(h1, W_rel2)
    acc2 = _sc_scatter(s1, dmr, t1, htab2, inv)
    return _tc_combine(root2, acc2)


# speedup vs baseline: 1.0988x; 1.0988x over previous
import functools

import jax
import jax.numpy as jnp
from jax import lax
from jax.experimental import pallas as pl
from jax.experimental.pallas import tpu as pltpu
from jax.experimental.pallas import tpu_sc as plsc

N = 10000
E = 320000
D = 128
R = 65
NC = 2
NS = 16
NT = NC * NS
KB = 80
EPT = E // NS
NBLK2 = EPT // KB
NBLK = NBLK2 // NC
CNTP = 651264
STRIPE = CNTP // NS
ZCH = 6784
NH = 5120
NPH = 5248
ZROWS = NPH // NS
WROWS = NH // NS

_mesh = plsc.VectorSubcoreMesh(core_axis_name="c", subcore_axis_name="s")


@functools.partial(
    pl.kernel,
    out_type=jax.ShapeDtypeStruct((NC, 1, CNTP), jnp.float32),
    mesh=_mesh,
    scratch_types=[
        pltpu.VMEM((NBLK, KB), jnp.int32),
        pltpu.VMEM((NBLK, KB), jnp.int32),
        pltpu.VMEM((NBLK, KB), jnp.int32),
        pltpu.VMEM((KB,), jnp.float32),
        pltpu.VMEM((ZCH,), jnp.float32),
        pltpu.VMEM_SHARED((CNTP,), jnp.float32),
    ],
)
def _sc_counts(d3, t3, out, dv, tv, civ, ones, zc, cnt_sh):
    c = lax.axis_index("c")
    s = lax.axis_index("s")
    wid = c * NS + s

    def fill_ones(i, carry):
        ones[pl.ds(i * 16, 16)] = jnp.full((16,), 1.0, jnp.float32)
        return carry

    lax.fori_loop(0, KB // 16, fill_ones, 0)

    def fill_z(i, carry):
        zc[pl.ds(i * 16, 16)] = jnp.zeros((16,), jnp.float32)
        return carry

    lax.fori_loop(0, ZCH // 16, fill_z, 0)

    base0 = s * STRIPE
    for k in range(STRIPE // ZCH):
        pltpu.sync_copy(zc, cnt_sh.at[pl.ds(base0 + k * ZCH, ZCH)])
    plsc.subcore_barrier()

    pltpu.sync_copy(d3.at[wid], dv)
    pltpu.sync_copy(t3.at[wid], tv)

    def mkci(i, carry):
        j = i // 5
        k = i % 5
        sl = pl.ds(k * 16, 16)
        civ[j, sl] = tv[j, sl] * N + dv[j, sl]
        return carry

    lax.fori_loop(0, NBLK * 5, mkci, 0)

    def scat(j, carry):
        pltpu.sync_copy(ones, cnt_sh.at[civ.at[j]], add=True)
        return carry

    lax.fori_loop(0, NBLK, scat, 0)
    plsc.subcore_barrier()

    pltpu.sync_copy(cnt_sh.at[pl.ds(base0, STRIPE)],
                    out.at[c, 0, pl.ds(base0, STRIPE)])


@functools.partial(
    pl.kernel,
    out_type=jax.ShapeDtypeStruct((NC, NH, D), jnp.float32),
    mesh=_mesh,
    scratch_types=[
        pltpu.VMEM((EPT,), jnp.int32),
        pltpu.VMEM((EPT,), jnp.int32),
        pltpu.VMEM((EPT,), jnp.int32),
        pltpu.VMEM((KB,), jnp.int32),
        pltpu.VMEM((KB,), jnp.int32),
        pltpu.VMEM((KB + 16,), jnp.float32),
        pltpu.VMEM((KB + 16,), jnp.float32),
        pltpu.VMEM((KB, D), jnp.float32),
        pltpu.VMEM((KB, D), jnp.float32),
        pltpu.VMEM_SHARED((NPH, D), jnp.float32),
        pltpu.SemaphoreType.DMA,
        pltpu.SemaphoreType.DMA,
        pltpu.SemaphoreType.DMA,
        pltpu.SemaphoreType.DMA,
        pltpu.SemaphoreType.DMA,
        pltpu.SemaphoreType.DMA,
    ],
)
def _sc_scatter(s1, dm, t1, htab, inv, out, d1, gv, civ, dvb0, dvb1,
                wvb0, wvb1, rows0, rows1, acc_sh,
                semg0, semg1, semw0, semw1, sems0, sems1):
    c = lax.axis_index("c")
    s = lax.axis_index("s")
    r0z = s * ZROWS
    r0w = s * WROWS

    pltpu.sync_copy(s1.at[s], gv)
    pltpu.sync_copy(dm.at[s], d1)
    pltpu.sync_copy(t1.at[s], civ)
    lo = c * NH

    def mkidx(i, carry):
        sl = pl.ds(i * 16, 16)
        tt = civ[sl]
        gv[sl] = tt * N + gv[sl]
        civ[sl] = tt * N + d1[sl]
        return carry

    lax.fori_loop(0, EPT // 16, mkidx, 0)

    def zrow(i, carry):
        rows0[i // 8, pl.ds((i % 8) * 16, 16)] = jnp.zeros((16,), jnp.float32)
        return carry

    lax.fori_loop(0, KB * 8, zrow, 0)
    for k in range(ZROWS // KB):
        pltpu.sync_copy(rows0, acc_sh.at[pl.ds(r0z + k * KB, KB)])
    pltpu.sync_copy(rows0.at[pl.ds(0, ZROWS % KB)],
                    acc_sh.at[pl.ds(r0z + (ZROWS // KB) * KB, ZROWS % KB)])
    plsc.subcore_barrier()

    bufs = ((rows0, wvb0, dvb0, semg0, semw0, sems0),
            (rows1, wvb1, dvb1, semg1, semw1, sems1))

    def _issue_gather(j, p):
        rw, wv, _, sg, sw, _ = bufs[p]
        pltpu.async_copy(htab.at[gv.at[pl.ds(j * KB, KB)]], rw, sg)
        pltpu.async_copy(inv.at[civ.at[pl.ds(j * KB, KB)]],
                         wv.at[pl.ds(0, KB)], sw)

    def _wait_gather(j, p):
        rw, wv, _, sg, sw, _ = bufs[p]
        pltpu.make_async_copy(htab.at[gv.at[pl.ds(j * KB, KB)]], rw, sg).wait()
        pltpu.make_async_copy(inv.at[civ.at[pl.ds(j * KB, KB)]],
                              wv.at[pl.ds(0, KB)], sw).wait()

    def _issue_scatter(j, p):
        rw, _, db, _, _, ss = bufs[p]
        pltpu.async_copy(rw, acc_sh.at[db], ss, add=True)

    def _wait_scatter(j, p):
        rw, _, db, _, _, ss = bufs[p]
        pltpu.make_async_copy(rw, acc_sh.at[db], ss).wait()

    def _mkdvb(j, p):
        db = bufs[p][2]
        for k in range(KB // 16):
            sl = pl.ds(j * KB + k * 16, 16)
            local = d1[sl] - lo
            ok = (local >= 0) & (local < NH)
            db[pl.ds(k * 16, 16)] = jnp.where(ok, local, NH)

    def _scale(j, p):
        rw, wv = bufs[p][0], bufs[p][1]

        def body(g, inner):
            wch = wv[pl.ds(g * 16, 16)]
            for l in range(16):
                w = wch[l]
                e = g * 16 + l
                for k in range(8):
                    sl = pl.ds(k * 16, 16)
                    rw[e, sl] = rw[e, sl] * w
            return inner

        lax.fori_loop(0, KB // 16, body, 0)

    _issue_gather(0, 0)
    _wait_gather(0, 0)
    _issue_gather(1, 1)
    _mkdvb(0, 0)
    _scale(0, 0)
    _issue_scatter(0, 0)

    def pair(j2, carry):
        j = 2 * j2 + 1
        _wait_gather(j, 1)
        _wait_scatter(j - 1, 0)
        _issue_gather(j + 1, 0)
        _mkdvb(j, 1)
        _scale(j, 1)
        _issue_scatter(j, 1)
        j = 2 * j2 + 2
        _wait_gather(j, 0)
        _wait_scatter(j - 1, 1)
        _issue_gather(j + 1, 1)
        _mkdvb(j, 0)
        _scale(j, 0)
        _issue_scatter(j, 0)
        return carry

    lax.fori_loop(0, NBLK2 // 2 - 1, pair, 0)
    j = NBLK2 - 1
    _wait_gather(j, 1)
    _wait_scatter(j - 1, 0)
    _mkdvb(j, 1)
    _scale(j, 1)
    _issue_scatter(j, 1)
    _wait_scatter(j, 1)
    plsc.subcore_barrier()
    pltpu.sync_copy(acc_sh.at[pl.ds(r0w, WROWS)],
                    out.at[c, pl.ds(r0w, WROWS)])


def _inv_body(c_ref, o_ref):
    o_ref[...] = 1.0 / jnp.maximum(c_ref[0] + c_ref[1], 1.0)


_tc_inv = pl.pallas_call(
    _inv_body,
    out_shape=jax.ShapeDtypeStruct((CNTP // 128, 128), jnp.float32),
)


def _root_body(h_ref, wr_ref, ws_ref, b_ref, o_ref):
    w = wr_ref[...] + ws_ref[...]
    o_ref[...] = jnp.dot(h_ref[...], w, preferred_element_type=jnp.float32) + b_ref[...]


_tc_root = pl.pallas_call(
    _root_body,
    out_shape=jax.ShapeDtypeStruct((N, D), jnp.float32),
)


def _h_body(h_ref, w_ref, o_ref):
    o_ref[...] = jnp.dot(h_ref[...], w_ref[0], preferred_element_type=jnp.float32)


_tc_h = pl.pallas_call(
    _h_body,
    grid=(5, R),
    in_specs=[
        pl.BlockSpec((N // 5, D), lambda i, r: (i, 0)),
        pl.BlockSpec((1, D, D), lambda i, r: (r, 0, 0)),
    ],
    out_specs=pl.BlockSpec((N // 5, D), lambda i, r: (r * 5 + i, 0)),
    out_shape=jax.ShapeDtypeStruct((R * N, D), jnp.float32),
)


def _comb_body(r_ref, a_ref, o_ref):
    agg = jnp.concatenate([a_ref[0], a_ref[1]], axis=0)[:N]
    o_ref[...] = jnp.maximum(r_ref[...] + agg, 0.0)


_tc_combine = pl.pallas_call(
    _comb_body,
    out_shape=jax.ShapeDtypeStruct((N, D), jnp.float32),
)


def kernel(x, edge_index, edge_type, W_rel1, W_root1, b1, W_self1,
           W_rel2, W_root2, b2, W_self2):
    s1 = edge_index[0].reshape(NS, EPT)
    dmr = edge_index[1].reshape(NS, EPT)
    t1 = edge_type.reshape(NS, EPT)

    cnt2 = _sc_counts(edge_index[1].reshape(NT, NBLK, KB),
                      edge_type.reshape(NT, NBLK, KB))
    inv = _tc_inv(cnt2.reshape(NC, CNTP // 128, 128)).reshape(CNTP)

    root1 = _tc_root(x, W_root1, W_self1, b1.reshape(1, D))
    htab1 = _tc_h(x, W_rel1)
    acc1 = _sc_scatter(s1, dmr, t1, htab1, inv)
    h1 = _tc_combine(root1, acc1)

    root2 = _tc_root(h1, W_root2, W_self2, b2.reshape(1, D))
    htab2 = _tc_h(h1, W_rel2)
    acc2 = _sc_scatter(s1, dmr, t1, htab2, inv)
    return _tc_combine(root2, acc2)
